# Initial kernel scaffold; baseline (speedup 1.0000x reference)
import jax
import jax.numpy as jnp
from jax.experimental import pallas as pl


def kernel(x, A_in, A_motif, coords, params):
    raise NotImplementedError("write your pallas kernel here")


# speedup vs baseline: 4.1232x; 4.1232x over previous
import jax
import jax.numpy as jnp
from jax.experimental import pallas as pl

N = 2048
DIN = 128
HID = 64
H2 = 2 * HID
C = 64
TK = 8
BR = 256
NB = N // BR
NEGINF = float("-inf")


def _rows_cols(i0):
    rows = i0 + jax.lax.broadcasted_iota(jnp.int32, (BR, 1), 0)
    cols = jax.lax.broadcasted_iota(jnp.int32, (BR, N), 1)
    return rows, cols


def _mhat_tile(vals_b, idx_b, vals_t, idx_t, rows, cols):
    sp = jnp.zeros((BR, N), jnp.float32)
    spT = jnp.zeros((BR, N), jnp.float32)
    for k in range(TK):
        sp = sp + jnp.where(cols == idx_b[:, k][:, None],
                            vals_b[:, k][:, None], 0.0)
        spT = spT + jnp.where(idx_t[k:k + 1, :] == rows,
                              vals_t[k:k + 1, :], 0.0)
    m = jnp.maximum(sp, spT)
    return jnp.where(cols == rows, 0.0, m)


def _gate(x, tau):
    s = jax.nn.sigmoid(x / tau)
    return jnp.clip(s * 1.2 - 0.1, 0.0, 1.0)


def _rowmax_first_argmax(v, cols):
    m = jnp.max(v, axis=1, keepdims=True)
    am = jnp.min(jnp.where(v == m, cols, N), axis=1, keepdims=True)
    return m, am


def _proj_body(x_ref, gw_ref, asrc_ref, adst_ref, gcnw_ref,
               hh_ref, sd_ref, xw_ref):
    xv = x_ref[...]
    hh = jnp.dot(xv, gw_ref[...], preferred_element_type=jnp.float32)
    hh_ref[...] = hh
    xw_ref[...] = jnp.dot(xv, gcnw_ref[...], preferred_element_type=jnp.float32)
    colsout = []
    for hd in range(2):
        hhd = hh[:, hd * HID:(hd + 1) * HID]
        colsout.append(jnp.dot(hhd, asrc_ref[hd, :][:, None],
                               preferred_element_type=jnp.float32))
    for hd in range(2):
        hhd = hh[:, hd * HID:(hd + 1) * HID]
        colsout.append(jnp.dot(hhd, adst_ref[hd, :][:, None],
                               preferred_element_type=jnp.float32))
    sd_ref[...] = jnp.concatenate(colsout, axis=1)


def _topk_body(am_ref, vals_ref, idx_ref):
    w = am_ref[...]
    cols = jax.lax.broadcasted_iota(jnp.int32, (BR, N), 1)
    vs, ins = [], []
    for _ in range(TK):
        m, am = _rowmax_first_argmax(w, cols)
        vs.append(m)
        ins.append(am)
        w = jnp.where(cols == am, NEGINF, w)
    vals_ref[...] = jnp.concatenate(vs, axis=1)
    idx_ref[...] = jnp.concatenate(ins, axis=1)


def _degm_body(vb_ref, ib_ref, vf_ref, if_ref, deg_ref):
    rows, cols = _rows_cols(pl.program_id(0) * BR)
    m = _mhat_tile(vb_ref[...], ib_ref[...], vf_ref[...], if_ref[...],
                   rows, cols)
    deg_ref[...] = jnp.sum(m, axis=1, keepdims=True)


def _gat_body(ain_ref, hh_ref, sdt_ref, sdb_ref, gout_ref):
    rows, cols = _rows_cols(pl.program_id(0) * BR)
    a = ain_ref[...]
    adjb = (a > 0) | (cols == rows)
    hh = hh_ref[...]
    sdt = sdt_ref[...]
    sdb = sdb_ref[...]
    outs = []
    for hd in range(2):
        hhd = hh[:, hd * HID:(hd + 1) * HID]
        s = sdt[hd:hd + 1, :]
        d = sdb[:, 2 + hd][:, None]
        e = d + s
        e = jnp.where(e >= 0, e, 0.2 * e)
        e = jnp.where(adjb, e, -1e9)
        m = jnp.max(e, axis=1, keepdims=True)
        p = jnp.exp(e - m)
        alpha = p / jnp.sum(p, axis=1, keepdims=True)
        outs.append(jnp.dot(alpha, hhd, preferred_element_type=jnp.float32))
    gout_ref[...] = jnp.concatenate(outs, axis=1)


def _gcn_body(vb_ref, ib_ref, vf_ref, if_ref, degf_ref, degb_ref, xw_ref,
              out_ref):
    rows, cols = _rows_cols(pl.program_id(0) * BR)
    m = _mhat_tile(vb_ref[...], ib_ref[...], vf_ref[...], if_ref[...],
                   rows, cols)
    degf = degf_ref[...]
    dinv_f = jnp.where(degf > 0, jax.lax.rsqrt(jnp.where(degf > 0, degf, 1.0)),
                       0.0)
    degb = degb_ref[...]
    dinv_b = jnp.where(degb > 0, jax.lax.rsqrt(jnp.where(degb > 0, degb, 1.0)),
                       0.0)
    dxw = dinv_f * xw_ref[...]
    out_ref[...] = dinv_b * jnp.dot(m, dxw, preferred_element_type=jnp.float32)


def _bn_elu(v, g, b):
    mu = jnp.mean(v, axis=0, keepdims=True)
    var = jnp.mean((v - mu) * (v - mu), axis=0, keepdims=True)
    z = (v - mu) / jnp.sqrt(var + 1e-5) * g + b
    return jnp.where(z > 0, z, jnp.exp(z) - 1.0)


def _combine_body(gout_ref, gcn_ref, gatb_ref, gcnb_ref, bnag_ref, bnab_ref,
                  bnmg_ref, bnmb_ref, scal_ref, wproj_ref, poolw_ref,
                  h_ref, vec4_ref, hp_ref):
    h_a = _bn_elu(gout_ref[...] + gatb_ref[...], bnag_ref[...], bnab_ref[...])
    h_m = _bn_elu(gcn_ref[...] + gcnb_ref[...], bnmg_ref[...], bnmb_ref[...])
    h = h_a + scal_ref[0, 5] * h_m
    h_ref[...] = h
    vec4_ref[...] = jnp.dot(h, wproj_ref[...],
                            preferred_element_type=jnp.float32)
    hp_ref[...] = jnp.dot(h, poolw_ref[...],
                          preferred_element_type=jnp.float32)


def _rewire_scores(a, coords_t, coords_b, m, vec4_t, vec4_b, rows, cols, scal):
    tau = scal[0, 0]
    w2r = scal[0, 3]
    rb = scal[0, 4]
    cxf = coords_t[0:1, :]
    cyf = coords_t[1:2, :]
    cxb = coords_b[:, 0][:, None]
    cyb = coords_b[:, 1][:, None]
    dist = jnp.abs(cxb - cxf) + jnp.abs(cyb - cyf)
    cand = (dist > 0) & (dist <= 2.0) & (a < 1e-6)
    arf = vec4_t[2:3, :]
    brf = vec4_t[3:4, :]
    arb = vec4_b[:, 2][:, None]
    brb = vec4_b[:, 3][:, None]
    base = m * w2r + rb
    l_ij = arb + brf + base
    l_ji = arf + brb + base
    za = _gate(jnp.where(rows < cols, l_ij, l_ji), tau)
    return cand, za


def _thr_body(ain_ref, coordsf_ref, coordsb_ref, vb_ref, ib_ref, vf_ref,
              if_ref, vec4f_ref, vec4b_ref, scal_ref, thr_ref):
    rows, cols = _rows_cols(pl.program_id(0) * BR)
    m = _mhat_tile(vb_ref[...], ib_ref[...], vf_ref[...], if_ref[...],
                   rows, cols)
    cand, za = _rewire_scores(ain_ref[...], coordsf_ref[...], coordsb_ref[...],
                              m, vec4f_ref[...], vec4b_ref[...], rows, cols,
                              scal_ref[...])
    neg = jnp.where(cand, za, NEGINF)
    m1, am = _rowmax_first_argmax(neg, cols)
    neg2 = jnp.where(cols == am, NEGINF, neg)
    thr_ref[...] = jnp.max(neg2, axis=1, keepdims=True)


def _refine_body(ain_ref, coordsf_ref, coordsb_ref, vb_ref, ib_ref, vf_ref,
                 if_ref, vec4f_ref, vec4b_ref, scal_ref, thrf_ref, thrb_ref,
                 aref_ref, deg_ref):
    rows, cols = _rows_cols(pl.program_id(0) * BR)
    a = ain_ref[...]
    scal = scal_ref[...]
    vec4_t = vec4f_ref[...]
    vec4_b = vec4b_ref[...]
    m = _mhat_tile(vb_ref[...], ib_ref[...], vf_ref[...], if_ref[...],
                   rows, cols)
    cand, za = _rewire_scores(a, coordsf_ref[...], coordsb_ref[...], m,
                              vec4_t, vec4_b, rows, cols, scal)
    zs = jnp.where(cand, za, 0.0)
    thr_b = thrb_ref[...]
    thr_f = thrf_ref[...]
    keep = cand & ((zs >= thr_b) | (zs >= thr_f))
    tau = scal[0, 0]
    w2p = scal[0, 1]
    pb = scal[0, 2]
    apf = vec4_t[0:1, :]
    bpf = vec4_t[1:2, :]
    apb = vec4_b[:, 0][:, None]
    bpb = vec4_b[:, 1][:, None]
    base = m * w2p + pb
    zp = _gate(jnp.where(rows < cols, apb + bpf + base, apf + bpb + base), tau)
    aref = a * zp + 0.5 * jnp.where(keep, zs, 0.0)
    aref_ref[...] = aref
    deg_ref[...] = jnp.sum(aref, axis=1, keepdims=True) + 1.0


def _pool_body(aref_ref, degf_ref, degb_ref, hpf_ref, hpb_ref, hb_ref,
               poolb_ref, s_ref, x_ref):
    degf = degf_ref[...]
    dinv_f = jnp.where(degf > 0, jax.lax.rsqrt(jnp.where(degf > 0, degf, 1.0)),
                       0.0)
    degb = degb_ref[...]
    dinv_b = jnp.where(degb > 0, jax.lax.rsqrt(jnp.where(degb > 0, degb, 1.0)),
                       0.0)
    dhp = dinv_f * hpf_ref[...]
    dhp_b = dinv_b * hpb_ref[...]
    row = jnp.dot(aref_ref[...], dhp, preferred_element_type=jnp.float32)
    logits = dinv_b * (row + dhp_b) + poolb_ref[...]
    m = jnp.max(logits, axis=1, keepdims=True)
    p = jnp.exp(logits - m)
    s = p / jnp.sum(p, axis=1, keepdims=True)
    s_ref[...] = s
    xc = jax.lax.dot_general(s, hb_ref[...], (((0,), (0,)), ((), ())),
                             preferred_element_type=jnp.float32)

    @pl.when(pl.program_id(0) == 0)
    def _():
        x_ref[...] = xc

    @pl.when(pl.program_id(0) != 0)
    def _():
        x_ref[...] += xc


def _coarse_body(aref_ref, sf_ref, sb_ref, ac_ref):
    t = jnp.dot(aref_ref[...], sf_ref[...], preferred_element_type=jnp.float32)
    acc = jax.lax.dot_general(sb_ref[...], t, (((0,), (0,)), ((), ())),
                              preferred_element_type=jnp.float32)

    @pl.when(pl.program_id(0) == 0)
    def _():
        ac_ref[...] = acc

    @pl.when(pl.program_id(0) != 0)
    def _():
        ac_ref[...] += acc


def _finish_body(ac_ref, out_ref):
    rows = jax.lax.broadcasted_iota(jnp.int32, (C, 1), 0)
    cols = jax.lax.broadcasted_iota(jnp.int32, (C, C), 1)
    diag = cols == rows
    w = jnp.where(diag, 0.0, ac_ref[...])
    sp = jnp.zeros((C, C), jnp.float32)
    spt = jnp.zeros((C, C), jnp.float32)
    for _ in range(TK):
        m = jnp.max(w, axis=1, keepdims=True)
        am = jnp.min(jnp.where(w == m, cols, C), axis=1, keepdims=True)
        sel = cols == am
        sp = jnp.where(sel, m, sp)
        spt = jnp.where(am[:, 0][None, :] == rows, m[:, 0][None, :], spt)
        w = jnp.where(sel, NEGINF, w)
    r = jnp.maximum(sp, spt)
    out_ref[...] = jnp.where(diag, 0.0, r)


def _f32(shape):
    return jax.ShapeDtypeStruct(shape, jnp.float32)


def _full(shape):
    return pl.BlockSpec(shape, lambda i: tuple(0 for _ in shape))


def _blk(shape):
    return pl.BlockSpec(shape, lambda i: (i,) + tuple(0 for _ in shape[1:]))


def kernel(x, A_in, A_motif, coords, params):
    p = params
    f32 = jnp.float32
    tau = jnp.maximum(p['tau'], 0.1).astype(f32)
    mu_sp = jax.nn.softplus(p['mu']).astype(f32)
    wp = p['prune_w']
    wr = p['rewire_w']
    scal = jnp.stack([tau, wp[2 * H2], p['prune_b'], wr[2 * H2],
                      p['rewire_b'], mu_sp, 0.0, 0.0]).reshape(1, 8)
    wproj = jnp.stack([wp[:H2], wp[H2:2 * H2], wr[:H2], wr[H2:2 * H2]],
                      axis=1)

    hh, sd, xw = pl.pallas_call(
        _proj_body,
        out_shape=(_f32((N, H2)), _f32((N, 4)), _f32((N, H2))),
    )(x, p['gat_W'], p['a_src'], p['a_dst'], p['gcn_W'])

    vals8, idx8 = pl.pallas_call(
        _topk_body,
        grid=(NB,),
        in_specs=[_blk((BR, N))],
        out_specs=(_blk((BR, TK)), _blk((BR, TK))),
        out_shape=(_f32((N, TK)), jax.ShapeDtypeStruct((N, TK), jnp.int32)),
    )(A_motif)

    vals_t = vals8.T
    idx_t = idx8.T
    sd_t = sd.T
    coords_t = coords.T

    degm = pl.pallas_call(
        _degm_body,
        grid=(NB,),
        in_specs=[_blk((BR, TK)), _blk((BR, TK)), _full((TK, N)),
                  _full((TK, N))],
        out_specs=_blk((BR, 1)),
        out_shape=_f32((N, 1)),
    )(vals8, idx8, vals_t, idx_t)

    gout = pl.pallas_call(
        _gat_body,
        grid=(NB,),
        in_specs=[_blk((BR, N)), _full((N, H2)), _full((4, N)), _blk((BR, 4))],
        out_specs=_blk((BR, H2)),
        out_shape=_f32((N, H2)),
    )(A_in, hh, sd_t, sd)

    gcn_pre = pl.pallas_call(
        _gcn_body,
        grid=(NB,),
        in_specs=[_blk((BR, TK)), _blk((BR, TK)), _full((TK, N)),
                  _full((TK, N)), _full((N, 1)), _blk((BR, 1)),
                  _full((N, H2))],
        out_specs=_blk((BR, H2)),
        out_shape=_f32((N, H2)),
    )(vals8, idx8, vals_t, idx_t, degm, degm, xw)

    h, vec4, hp = pl.pallas_call(
        _combine_body,
        out_shape=(_f32((N, H2)), _f32((N, 4)), _f32((N, C))),
    )(gout, gcn_pre, p['gat_b'].reshape(1, H2), p['gcn_b'].reshape(1, H2),
      p['bnA_g'].reshape(1, H2), p['bnA_b'].reshape(1, H2),
      p['bnM_g'].reshape(1, H2), p['bnM_b'].reshape(1, H2),
      scal, wproj, p['pool_W'])

    vec4_t = vec4.T

    thr = pl.pallas_call(
        _thr_body,
        grid=(NB,),
        in_specs=[_blk((BR, N)), _full((2, N)), _blk((BR, 2)),
                  _blk((BR, TK)), _blk((BR, TK)), _full((TK, N)),
                  _full((TK, N)), _full((4, N)), _blk((BR, 4)),
                  _full((1, 8))],
        out_specs=_blk((BR, 1)),
        out_shape=_f32((N, 1)),
    )(A_in, coords_t, coords, vals8, idx8, vals_t, idx_t, vec4_t, vec4, scal)

    thr_t = thr.T

    aref, degr = pl.pallas_call(
        _refine_body,
        grid=(NB,),
        in_specs=[_blk((BR, N)), _full((2, N)), _blk((BR, 2)),
                  _blk((BR, TK)), _blk((BR, TK)), _full((TK, N)),
                  _full((TK, N)), _full((4, N)), _blk((BR, 4)),
                  _full((1, 8)), _full((1, N)), _blk((BR, 1))],
        out_specs=(_blk((BR, N)), _blk((BR, 1))),
        out_shape=(_f32((N, N)), _f32((N, 1))),
    )(A_in, coords_t, coords, vals8, idx8, vals_t, idx_t, vec4_t, vec4, scal,
      thr_t, thr)

    s_mat, x_coarse = pl.pallas_call(
        _pool_body,
        grid=(NB,),
        in_specs=[_blk((BR, N)), _full((N, 1)), _blk((BR, 1)),
                  _full((N, C)), _blk((BR, C)), _blk((BR, H2)),
                  _full((1, C))],
        out_specs=(_blk((BR, C)), _full((C, H2))),
        out_shape=(_f32((N, C)), _f32((C, H2))),
    )(aref, degr, degr, hp, hp, h, p['pool_b'].reshape(1, C))

    ac = pl.pallas_call(
        _coarse_body,
        grid=(NB,),
        in_specs=[_blk((BR, N)), _full((N, C)), _blk((BR, C))],
        out_specs=_full((C, C)),
        out_shape=_f32((C, C)),
    )(aref, s_mat, s_mat)

    a_coarse = pl.pallas_call(
        _finish_body,
        out_shape=_f32((C, C)),
    )(ac)

    return x_coarse, a_coarse


# speedup vs baseline: 4.3659x; 1.0589x over previous
import jax
import jax.numpy as jnp
from jax.experimental import pallas as pl
from jax.experimental.pallas import tpu as pltpu

N = 2048
DIN = 128
HID = 64
H2 = 2 * HID
C = 64
TK = 8
BR = 256
NB = N // BR
NEGINF = float("-inf")


def _rows_cols(i0):
    rows = i0 + jax.lax.broadcasted_iota(jnp.int32, (BR, 1), 0)
    cols = jax.lax.broadcasted_iota(jnp.int32, (BR, N), 1)
    return rows, cols


def _mhat_tile(vals_b, idx_b, vals_t, idx_t, rows, cols):
    sp = jnp.zeros((BR, N), jnp.float32)
    spT = jnp.zeros((BR, N), jnp.float32)
    for k in range(TK):
        sp = sp + jnp.where(cols == idx_b[:, k][:, None],
                            vals_b[:, k][:, None], 0.0)
        spT = spT + jnp.where(idx_t[k:k + 1, :] == rows,
                              vals_t[k:k + 1, :], 0.0)
    m = jnp.maximum(sp, spT)
    return jnp.where(cols == rows, 0.0, m)


def _gate(x, tau):
    s = jax.nn.sigmoid(x / tau)
    return jnp.clip(s * 1.2 - 0.1, 0.0, 1.0)


def _rowmax_first_argmax(v, cols):
    m = jnp.max(v, axis=1, keepdims=True)
    am = jnp.min(jnp.where(v == m, cols, N), axis=1, keepdims=True)
    return m, am


def _topk_proj_body(am_ref, x_ref, gw_ref, asrc_ref, adst_ref, gcnw_ref,
                    vals_ref, idx_ref, hh_ref, sd_ref, xw_ref):
    w = am_ref[...]
    cols = jax.lax.broadcasted_iota(jnp.int32, (BR, N), 1)
    vs, ins = [], []
    for _ in range(TK):
        m, am = _rowmax_first_argmax(w, cols)
        vs.append(m)
        ins.append(am)
        w = jnp.where(cols == am, NEGINF, w)
    vals_ref[...] = jnp.concatenate(vs, axis=1)
    idx_ref[...] = jnp.concatenate(ins, axis=1)

    @pl.when(pl.program_id(0) == 0)
    def _():
        xv = x_ref[...]
        hh = jnp.dot(xv, gw_ref[...], preferred_element_type=jnp.float32)
        hh_ref[...] = hh
        xw_ref[...] = jnp.dot(xv, gcnw_ref[...],
                              preferred_element_type=jnp.float32)
        colsout = []
        for hd in range(2):
            hhd = hh[:, hd * HID:(hd + 1) * HID]
            colsout.append(jnp.dot(hhd, asrc_ref[hd, :][:, None],
                                   preferred_element_type=jnp.float32))
        for hd in range(2):
            hhd = hh[:, hd * HID:(hd + 1) * HID]
            colsout.append(jnp.dot(hhd, adst_ref[hd, :][:, None],
                                   preferred_element_type=jnp.float32))
        sd_ref[...] = jnp.concatenate(colsout, axis=1)


def _gat_body(ain_ref, hh_ref, sdt_ref, sdb_ref, vb_ref, ib_ref, vt_ref,
              it_ref, gout_ref, degm_ref):
    rows, cols = _rows_cols(pl.program_id(0) * BR)
    m_hat = _mhat_tile(vb_ref[...], ib_ref[...], vt_ref[...], it_ref[...],
                       rows, cols)
    degm_ref[...] = jnp.sum(m_hat, axis=1, keepdims=True)
    a = ain_ref[...]
    adjb = (a > 0) | (cols == rows)
    hh = hh_ref[...]
    sdt = sdt_ref[...]
    sdb = sdb_ref[...]
    outs = []
    for hd in range(2):
        hhd = hh[:, hd * HID:(hd + 1) * HID]
        s = sdt[hd:hd + 1, :]
        d = sdb[:, 2 + hd][:, None]
        e = d + s
        e = jnp.where(e >= 0, e, 0.2 * e)
        e = jnp.where(adjb, e, -1e9)
        m = jnp.max(e, axis=1, keepdims=True)
        p = jnp.exp(e - m)
        alpha = p / jnp.sum(p, axis=1, keepdims=True)
        outs.append(jnp.dot(alpha, hhd, preferred_element_type=jnp.float32))
    gout_ref[...] = jnp.concatenate(outs, axis=1)


def _gcn_body(vb_ref, ib_ref, vf_ref, if_ref, degf_ref, degb_ref, xw_ref,
              out_ref):
    rows, cols = _rows_cols(pl.program_id(0) * BR)
    m = _mhat_tile(vb_ref[...], ib_ref[...], vf_ref[...], if_ref[...],
                   rows, cols)
    degf = degf_ref[...]
    dinv_f = jnp.where(degf > 0, jax.lax.rsqrt(jnp.where(degf > 0, degf, 1.0)),
                       0.0)
    degb = degb_ref[...]
    dinv_b = jnp.where(degb > 0, jax.lax.rsqrt(jnp.where(degb > 0, degb, 1.0)),
                       0.0)
    dxw = dinv_f * xw_ref[...]
    out_ref[...] = dinv_b * jnp.dot(m, dxw, preferred_element_type=jnp.float32)


def _bn_elu(v, g, b):
    mu = jnp.mean(v, axis=0, keepdims=True)
    var = jnp.mean((v - mu) * (v - mu), axis=0, keepdims=True)
    z = (v - mu) / jnp.sqrt(var + 1e-5) * g + b
    return jnp.where(z > 0, z, jnp.exp(z) - 1.0)


def _combine_body(gout_ref, gcn_ref, gatb_ref, gcnb_ref, bnag_ref, bnab_ref,
                  bnmg_ref, bnmb_ref, scal_ref, wproj_ref, poolw_ref,
                  h_ref, vec4_ref, hp_ref):
    h_a = _bn_elu(gout_ref[...] + gatb_ref[...], bnag_ref[...], bnab_ref[...])
    h_m = _bn_elu(gcn_ref[...] + gcnb_ref[...], bnmg_ref[...], bnmb_ref[...])
    h = h_a + scal_ref[0, 5] * h_m
    h_ref[...] = h
    vec4_ref[...] = jnp.dot(h, wproj_ref[...],
                            preferred_element_type=jnp.float32)
    hp_ref[...] = jnp.dot(h, poolw_ref[...],
                          preferred_element_type=jnp.float32)


def _rewire_scores(a, coords_t, coords_b, m, vec4_t, vec4_b, rows, cols, scal):
    tau = scal[0, 0]
    w2r = scal[0, 3]
    rb = scal[0, 4]
    cxf = coords_t[0:1, :]
    cyf = coords_t[1:2, :]
    cxb = coords_b[:, 0][:, None]
    cyb = coords_b[:, 1][:, None]
    dist = jnp.abs(cxb - cxf) + jnp.abs(cyb - cyf)
    cand = (dist > 0) & (dist <= 2.0) & (a < 1e-6)
    arf = vec4_t[2:3, :]
    brf = vec4_t[3:4, :]
    arb = vec4_b[:, 2][:, None]
    brb = vec4_b[:, 3][:, None]
    base = m * w2r + rb
    l_ij = arb + brf + base
    l_ji = arf + brb + base
    return cand, jnp.where(rows < cols, l_ij, l_ji)


def _thr_body(ain_ref, coordsf_ref, coordsb_ref, vb_ref, ib_ref, vf_ref,
              if_ref, vec4f_ref, vec4b_ref, scal_ref, thr_ref):
    rows, cols = _rows_cols(pl.program_id(0) * BR)
    m = _mhat_tile(vb_ref[...], ib_ref[...], vf_ref[...], if_ref[...],
                   rows, cols)
    cand, lr = _rewire_scores(ain_ref[...], coordsf_ref[...], coordsb_ref[...],
                              m, vec4f_ref[...], vec4b_ref[...], rows, cols,
                              scal_ref[...])
    neg = jnp.where(cand, lr, NEGINF)
    m1, am = _rowmax_first_argmax(neg, cols)
    neg2 = jnp.where(cols == am, NEGINF, neg)
    l2 = jnp.max(neg2, axis=1, keepdims=True)
    thr_ref[...] = jnp.where(l2 == NEGINF, NEGINF, _gate(l2, scal_ref[0, 0]))


def _refine_body(ain_ref, coordsf_ref, coordsb_ref, vb_ref, ib_ref, vf_ref,
                 if_ref, vec4f_ref, vec4b_ref, scal_ref, thrf_ref, thrb_ref,
                 aref_ref, deg_ref):
    rows, cols = _rows_cols(pl.program_id(0) * BR)
    a = ain_ref[...]
    scal = scal_ref[...]
    vec4_t = vec4f_ref[...]
    vec4_b = vec4b_ref[...]
    m = _mhat_tile(vb_ref[...], ib_ref[...], vf_ref[...], if_ref[...],
                   rows, cols)
    cand, lr = _rewire_scores(a, coordsf_ref[...], coordsb_ref[...], m,
                              vec4_t, vec4_b, rows, cols, scal)
    zs = jnp.where(cand, _gate(lr, scal[0, 0]), 0.0)
    thr_b = thrb_ref[...]
    thr_f = thrf_ref[...]
    keep = cand & ((zs >= thr_b) | (zs >= thr_f))
    tau = scal[0, 0]
    w2p = scal[0, 1]
    pb = scal[0, 2]
    apf = vec4_t[0:1, :]
    bpf = vec4_t[1:2, :]
    apb = vec4_b[:, 0][:, None]
    bpb = vec4_b[:, 1][:, None]
    base = m * w2p + pb
    zp = _gate(jnp.where(rows < cols, apb + bpf + base, apf + bpb + base), tau)
    aref = a * zp + 0.5 * jnp.where(keep, zs, 0.0)
    aref_ref[...] = aref
    deg_ref[...] = jnp.sum(aref, axis=1, keepdims=True) + 1.0


def _pool_body(aref_ref, degf_ref, degb_ref, hpf_ref, hpb_ref, hb_ref,
               poolb_ref, s_ref, x_ref):
    degf = degf_ref[...]
    dinv_f = jnp.where(degf > 0, jax.lax.rsqrt(jnp.where(degf > 0, degf, 1.0)),
                       0.0)
    degb = degb_ref[...]
    dinv_b = jnp.where(degb > 0, jax.lax.rsqrt(jnp.where(degb > 0, degb, 1.0)),
                       0.0)
    dhp = dinv_f * hpf_ref[...]
    dhp_b = dinv_b * hpb_ref[...]
    row = jnp.dot(aref_ref[...], dhp, preferred_element_type=jnp.float32)
    logits = dinv_b * (row + dhp_b) + poolb_ref[...]
    m = jnp.max(logits, axis=1, keepdims=True)
    p = jnp.exp(logits - m)
    s = p / jnp.sum(p, axis=1, keepdims=True)
    s_ref[...] = s
    xc = jax.lax.dot_general(s, hb_ref[...], (((0,), (0,)), ((), ())),
                             preferred_element_type=jnp.float32)

    @pl.when(pl.program_id(0) == 0)
    def _():
        x_ref[...] = xc

    @pl.when(pl.program_id(0) != 0)
    def _():
        x_ref[...] += xc


def _coarse_body(aref_ref, sf_ref, sb_ref, out_ref, ac_ref):
    t = jnp.dot(aref_ref[...], sf_ref[...], preferred_element_type=jnp.float32)
    acc = jax.lax.dot_general(sb_ref[...], t, (((0,), (0,)), ((), ())),
                              preferred_element_type=jnp.float32)

    @pl.when(pl.program_id(0) == 0)
    def _():
        ac_ref[...] = acc

    @pl.when(pl.program_id(0) != 0)
    def _():
        ac_ref[...] += acc

    @pl.when(pl.program_id(0) == NB - 1)
    def _():
        rows = jax.lax.broadcasted_iota(jnp.int32, (C, 1), 0)
        cols = jax.lax.broadcasted_iota(jnp.int32, (C, C), 1)
        diag = cols == rows
        w = jnp.where(diag, 0.0, ac_ref[...])
        sp = jnp.zeros((C, C), jnp.float32)
        spt = jnp.zeros((C, C), jnp.float32)
        for _ in range(TK):
            m = jnp.max(w, axis=1, keepdims=True)
            am = jnp.min(jnp.where(w == m, cols, C), axis=1, keepdims=True)
            sel = cols == am
            sp = jnp.where(sel, m, sp)
            spt = jnp.where(am[:, 0][None, :] == rows, m[:, 0][None, :], spt)
            w = jnp.where(sel, NEGINF, w)
        r = jnp.maximum(sp, spt)
        out_ref[...] = jnp.where(diag, 0.0, r)


def _f32(shape):
    return jax.ShapeDtypeStruct(shape, jnp.float32)


def _full(shape):
    return pl.BlockSpec(shape, lambda i: tuple(0 for _ in shape))


def _blk(shape):
    return pl.BlockSpec(shape, lambda i: (i,) + tuple(0 for _ in shape[1:]))


def kernel(x, A_in, A_motif, coords, params):
    p = params
    f32 = jnp.float32
    tau = jnp.maximum(p['tau'], 0.1).astype(f32)
    mu_sp = jax.nn.softplus(p['mu']).astype(f32)
    wp = p['prune_w']
    wr = p['rewire_w']
    scal = jnp.stack([tau, wp[2 * H2], p['prune_b'], wr[2 * H2],
                      p['rewire_b'], mu_sp, 0.0, 0.0]).reshape(1, 8)
    wproj = jnp.stack([wp[:H2], wp[H2:2 * H2], wr[:H2], wr[H2:2 * H2]],
                      axis=1)

    vals8, idx8, hh, sd, xw = pl.pallas_call(
        _topk_proj_body,
        grid=(NB,),
        in_specs=[_blk((BR, N)), _full((N, DIN)), _full((DIN, H2)),
                  _full((2, HID)), _full((2, HID)), _full((DIN, H2))],
        out_specs=(_blk((BR, TK)), _blk((BR, TK)), _full((N, H2)),
                   _full((N, 4)), _full((N, H2))),
        out_shape=(_f32((N, TK)), jax.ShapeDtypeStruct((N, TK), jnp.int32),
                   _f32((N, H2)), _f32((N, 4)), _f32((N, H2))),
    )(A_motif, x, p['gat_W'], p['a_src'], p['a_dst'], p['gcn_W'])

    vals_t = vals8.T
    idx_t = idx8.T
    sd_t = sd.T
    coords_t = coords.T

    gout, degm = pl.pallas_call(
        _gat_body,
        grid=(NB,),
        in_specs=[_blk((BR, N)), _full((N, H2)), _full((4, N)), _blk((BR, 4)),
                  _blk((BR, TK)), _blk((BR, TK)), _full((TK, N)),
                  _full((TK, N))],
        out_specs=(_blk((BR, H2)), _blk((BR, 1))),
        out_shape=(_f32((N, H2)), _f32((N, 1))),
    )(A_in, hh, sd_t, sd, vals8, idx8, vals_t, idx_t)

    gcn_pre = pl.pallas_call(
        _gcn_body,
        grid=(NB,),
        in_specs=[_blk((BR, TK)), _blk((BR, TK)), _full((TK, N)),
                  _full((TK, N)), _full((N, 1)), _blk((BR, 1)),
                  _full((N, H2))],
        out_specs=_blk((BR, H2)),
        out_shape=_f32((N, H2)),
    )(vals8, idx8, vals_t, idx_t, degm, degm, xw)

    h, vec4, hp = pl.pallas_call(
        _combine_body,
        out_shape=(_f32((N, H2)), _f32((N, 4)), _f32((N, C))),
    )(gout, gcn_pre, p['gat_b'].reshape(1, H2), p['gcn_b'].reshape(1, H2),
      p['bnA_g'].reshape(1, H2), p['bnA_b'].reshape(1, H2),
      p['bnM_g'].reshape(1, H2), p['bnM_b'].reshape(1, H2),
      scal, wproj, p['pool_W'])

    vec4_t = vec4.T

    thr = pl.pallas_call(
        _thr_body,
        grid=(NB,),
        in_specs=[_blk((BR, N)), _full((2, N)), _blk((BR, 2)),
                  _blk((BR, TK)), _blk((BR, TK)), _full((TK, N)),
                  _full((TK, N)), _full((4, N)), _blk((BR, 4)),
                  _full((1, 8))],
        out_specs=_blk((BR, 1)),
        out_shape=_f32((N, 1)),
    )(A_in, coords_t, coords, vals8, idx8, vals_t, idx_t, vec4_t, vec4, scal)

    thr_t = thr.T

    aref, degr = pl.pallas_call(
        _refine_body,
        grid=(NB,),
        in_specs=[_blk((BR, N)), _full((2, N)), _blk((BR, 2)),
                  _blk((BR, TK)), _blk((BR, TK)), _full((TK, N)),
                  _full((TK, N)), _full((4, N)), _blk((BR, 4)),
                  _full((1, 8)), _full((1, N)), _blk((BR, 1))],
        out_specs=(_blk((BR, N)), _blk((BR, 1))),
        out_shape=(_f32((N, N)), _f32((N, 1))),
    )(A_in, coords_t, coords, vals8, idx8, vals_t, idx_t, vec4_t, vec4, scal,
      thr_t, thr)

    s_mat, x_coarse = pl.pallas_call(
        _pool_body,
        grid=(NB,),
        in_specs=[_blk((BR, N)), _full((N, 1)), _blk((BR, 1)),
                  _full((N, C)), _blk((BR, C)), _blk((BR, H2)),
                  _full((1, C))],
        out_specs=(_blk((BR, C)), _full((C, H2))),
        out_shape=(_f32((N, C)), _f32((C, H2))),
    )(aref, degr, degr, hp, hp, h, p['pool_b'].reshape(1, C))

    a_coarse = pl.pallas_call(
        _coarse_body,
        grid=(NB,),
        in_specs=[_blk((BR, N)), _full((N, C)), _blk((BR, C))],
        out_specs=_full((C, C)),
        out_shape=_f32((C, C)),
        scratch_shapes=[pltpu.VMEM((C, C), jnp.float32)],
    )(aref, s_mat, s_mat)

    return x_coarse, a_coarse


# speedup vs baseline: 4.4577x; 1.0210x over previous
import jax
import jax.numpy as jnp
from jax.experimental import pallas as pl
from jax.experimental.pallas import tpu as pltpu

N = 2048
DIN = 128
HID = 64
H2 = 2 * HID
C = 64
TK = 8
BR = 512
NB = N // BR
NEGINF = float("-inf")


def _rows_cols(i0):
    rows = i0 + jax.lax.broadcasted_iota(jnp.int32, (BR, 1), 0)
    cols = jax.lax.broadcasted_iota(jnp.int32, (BR, N), 1)
    return rows, cols


def _mhat_tile(vals_b, idx_b, vals_t, idx_t, rows, cols):
    sp = jnp.zeros((BR, N), jnp.float32)
    spT = jnp.zeros((BR, N), jnp.float32)
    for k in range(TK):
        sp = sp + jnp.where(cols == idx_b[:, k][:, None],
                            vals_b[:, k][:, None], 0.0)
        spT = spT + jnp.where(idx_t[k:k + 1, :] == rows,
                              vals_t[k:k + 1, :], 0.0)
    m = jnp.maximum(sp, spT)
    return jnp.where(cols == rows, 0.0, m)


def _gate(x, tau):
    s = jax.nn.sigmoid(x / tau)
    return jnp.clip(s * 1.2 - 0.1, 0.0, 1.0)


def _rowmax_first_argmax(v, cols):
    m = jnp.max(v, axis=1, keepdims=True)
    am = jnp.min(jnp.where(v == m, cols, N), axis=1, keepdims=True)
    return m, am


def _topk_proj_body(am_ref, x_ref, gw_ref, asrc_ref, adst_ref, gcnw_ref,
                    vals_ref, idx_ref, hh_ref, sd_ref, sdt_ref, xw_ref):
    w = am_ref[...]
    cols = jax.lax.broadcasted_iota(jnp.int32, (BR, N), 1)
    vs, ins = [], []
    for _ in range(TK):
        m, am = _rowmax_first_argmax(w, cols)
        vs.append(m)
        ins.append(am)
        w = jnp.where(cols == am, NEGINF, w)
    vals_ref[...] = jnp.concatenate(vs, axis=1)
    idx_ref[...] = jnp.concatenate(ins, axis=1)

    @pl.when(pl.program_id(0) == 0)
    def _():
        xv = x_ref[...]
        hh = jnp.dot(xv, gw_ref[...], preferred_element_type=jnp.float32)
        hh_ref[...] = hh
        xw_ref[...] = jnp.dot(xv, gcnw_ref[...],
                              preferred_element_type=jnp.float32)
        dcols = []
        srows = []
        for hd in range(2):
            hhd = hh[:, hd * HID:(hd + 1) * HID]
            srows.append(jax.lax.dot_general(
                asrc_ref[hd:hd + 1, :], hhd, (((1,), (1,)), ((), ())),
                preferred_element_type=jnp.float32))
            dcols.append(jnp.dot(hhd, adst_ref[hd, :][:, None],
                                 preferred_element_type=jnp.float32))
        sd_ref[...] = jnp.concatenate(dcols, axis=1)
        sdt_ref[...] = jnp.concatenate(srows, axis=0)


def _gat_body(ain_ref, hh_ref, sdt_ref, sdb_ref, vb_ref, ib_ref, vt_ref,
              it_ref, gout_ref, degm_ref):
    rows, cols = _rows_cols(pl.program_id(0) * BR)
    m_hat = _mhat_tile(vb_ref[...], ib_ref[...], vt_ref[...], it_ref[...],
                       rows, cols)
    degm_ref[...] = jnp.sum(m_hat, axis=1, keepdims=True)
    a = ain_ref[...]
    adjb = (a > 0) | (cols == rows)
    hh = hh_ref[...]
    sdt = sdt_ref[...]
    sdb = sdb_ref[...]
    outs = []
    for hd in range(2):
        hhd = hh[:, hd * HID:(hd + 1) * HID]
        s = sdt[hd:hd + 1, :]
        d = sdb[:, hd][:, None]
        e = d + s
        e = jnp.where(e >= 0, e, 0.2 * e)
        e = jnp.where(adjb, e, -1e9)
        m = jnp.max(e, axis=1, keepdims=True)
        p = jnp.exp(e - m)
        alpha = p / jnp.sum(p, axis=1, keepdims=True)
        outs.append(jnp.dot(alpha, hhd, preferred_element_type=jnp.float32))
    gout_ref[...] = jnp.concatenate(outs, axis=1)


def _gcn_body(vb_ref, ib_ref, vf_ref, if_ref, degf_ref, degb_ref, xw_ref,
              out_ref):
    rows, cols = _rows_cols(pl.program_id(0) * BR)
    m = _mhat_tile(vb_ref[...], ib_ref[...], vf_ref[...], if_ref[...],
                   rows, cols)
    degf = degf_ref[...]
    dinv_f = jnp.where(degf > 0, jax.lax.rsqrt(jnp.where(degf > 0, degf, 1.0)),
                       0.0)
    degb = degb_ref[...]
    dinv_b = jnp.where(degb > 0, jax.lax.rsqrt(jnp.where(degb > 0, degb, 1.0)),
                       0.0)
    dxw = dinv_f * xw_ref[...]
    out_ref[...] = dinv_b * jnp.dot(m, dxw, preferred_element_type=jnp.float32)


def _bn_elu(v, g, b):
    mu = jnp.mean(v, axis=0, keepdims=True)
    var = jnp.mean((v - mu) * (v - mu), axis=0, keepdims=True)
    z = (v - mu) / jnp.sqrt(var + 1e-5) * g + b
    return jnp.where(z > 0, z, jnp.exp(z) - 1.0)


def _combine_body(gout_ref, gcn_ref, gatb_ref, gcnb_ref, bnag_ref, bnab_ref,
                  bnmg_ref, bnmb_ref, scal_ref, wproj_ref, poolw_ref,
                  h_ref, vec4_ref, hp_ref):
    h_a = _bn_elu(gout_ref[...] + gatb_ref[...], bnag_ref[...], bnab_ref[...])
    h_m = _bn_elu(gcn_ref[...] + gcnb_ref[...], bnmg_ref[...], bnmb_ref[...])
    h = h_a + scal_ref[0, 5] * h_m
    h_ref[...] = h
    vec4_ref[...] = jax.lax.dot_general(
        h, wproj_ref[...], (((1,), (1,)), ((), ())),
        preferred_element_type=jnp.float32)
    hp_ref[...] = jnp.dot(h, poolw_ref[...],
                          preferred_element_type=jnp.float32)


def _rewire_scores(a, coords_t, coords_b, m, vec4_t, vec4_b, rows, cols, scal):
    tau = scal[0, 0]
    w2r = scal[0, 3]
    rb = scal[0, 4]
    cxf = coords_t[0:1, :]
    cyf = coords_t[1:2, :]
    cxb = coords_b[:, 0][:, None]
    cyb = coords_b[:, 1][:, None]
    dist = jnp.abs(cxb - cxf) + jnp.abs(cyb - cyf)
    cand = (dist > 0) & (dist <= 2.0) & (a < 1e-6)
    arf = vec4_t[2:3, :]
    brf = vec4_t[3:4, :]
    arb = vec4_b[:, 2][:, None]
    brb = vec4_b[:, 3][:, None]
    base = m * w2r + rb
    l_ij = arb + brf + base
    l_ji = arf + brb + base
    return cand, jnp.where(rows < cols, l_ij, l_ji)


def _thr_body(ain_ref, coordsf_ref, coordsb_ref, vb_ref, ib_ref, vf_ref,
              if_ref, vec4f_ref, vec4b_ref, scal_ref, thr_ref):
    rows, cols = _rows_cols(pl.program_id(0) * BR)
    m = _mhat_tile(vb_ref[...], ib_ref[...], vf_ref[...], if_ref[...],
                   rows, cols)
    cand, lr = _rewire_scores(ain_ref[...], coordsf_ref[...], coordsb_ref[...],
                              m, vec4f_ref[...], vec4b_ref[...], rows, cols,
                              scal_ref[...])
    neg = jnp.where(cand, lr, NEGINF)
    m1, am = _rowmax_first_argmax(neg, cols)
    neg2 = jnp.where(cols == am, NEGINF, neg)
    l2 = jnp.max(neg2, axis=1, keepdims=True)
    thr_ref[...] = jnp.where(l2 == NEGINF, NEGINF, _gate(l2, scal_ref[0, 0]))


def _refine_body(ain_ref, coordsf_ref, coordsb_ref, vb_ref, ib_ref, vf_ref,
                 if_ref, vec4f_ref, vec4b_ref, scal_ref, thrf_ref, thrb_ref,
                 aref_ref, deg_ref):
    rows, cols = _rows_cols(pl.program_id(0) * BR)
    a = ain_ref[...]
    scal = scal_ref[...]
    vec4_t = vec4f_ref[...]
    vec4_b = vec4b_ref[...]
    m = _mhat_tile(vb_ref[...], ib_ref[...], vf_ref[...], if_ref[...],
                   rows, cols)
    cand, lr = _rewire_scores(a, coordsf_ref[...], coordsb_ref[...], m,
                              vec4_t, vec4_b, rows, cols, scal)
    zs = jnp.where(cand, _gate(lr, scal[0, 0]), 0.0)
    thr_b = thrb_ref[...]
    thr_f = thrf_ref[...]
    keep = cand & ((zs >= thr_b) | (zs >= thr_f))
    tau = scal[0, 0]
    w2p = scal[0, 1]
    pb = scal[0, 2]
    apf = vec4_t[0:1, :]
    bpf = vec4_t[1:2, :]
    apb = vec4_b[:, 0][:, None]
    bpb = vec4_b[:, 1][:, None]
    base = m * w2p + pb
    zp = _gate(jnp.where(rows < cols, apb + bpf + base, apf + bpb + base), tau)
    aref = a * zp + 0.5 * jnp.where(keep, zs, 0.0)
    aref_ref[...] = aref
    deg_ref[...] = jnp.sum(aref, axis=1, keepdims=True) + 1.0


def _pool_body(aref_ref, degf_ref, degb_ref, hpf_ref, hpb_ref, hb_ref,
               poolb_ref, s_ref, x_ref):
    degf = degf_ref[...]
    dinv_f = jnp.where(degf > 0, jax.lax.rsqrt(jnp.where(degf > 0, degf, 1.0)),
                       0.0)
    degb = degb_ref[...]
    dinv_b = jnp.where(degb > 0, jax.lax.rsqrt(jnp.where(degb > 0, degb, 1.0)),
                       0.0)
    dhp = dinv_f * hpf_ref[...]
    dhp_b = dinv_b * hpb_ref[...]
    row = jnp.dot(aref_ref[...], dhp, preferred_element_type=jnp.float32)
    logits = dinv_b * (row + dhp_b) + poolb_ref[...]
    m = jnp.max(logits, axis=1, keepdims=True)
    p = jnp.exp(logits - m)
    s = p / jnp.sum(p, axis=1, keepdims=True)
    s_ref[...] = s
    xc = jax.lax.dot_general(s, hb_ref[...], (((0,), (0,)), ((), ())),
                             preferred_element_type=jnp.float32)

    @pl.when(pl.program_id(0) == 0)
    def _():
        x_ref[...] = xc

    @pl.when(pl.program_id(0) != 0)
    def _():
        x_ref[...] += xc


def _coarse_body(aref_ref, sf_ref, sb_ref, out_ref, ac_ref):
    t = jnp.dot(aref_ref[...], sf_ref[...], preferred_element_type=jnp.float32)
    acc = jax.lax.dot_general(sb_ref[...], t, (((0,), (0,)), ((), ())),
                              preferred_element_type=jnp.float32)

    @pl.when(pl.program_id(0) == 0)
    def _():
        ac_ref[...] = acc

    @pl.when(pl.program_id(0) != 0)
    def _():
        ac_ref[...] += acc

    @pl.when(pl.program_id(0) == NB - 1)
    def _():
        rows = jax.lax.broadcasted_iota(jnp.int32, (C, 1), 0)
        cols = jax.lax.broadcasted_iota(jnp.int32, (C, C), 1)
        diag = cols == rows
        w = jnp.where(diag, 0.0, ac_ref[...])
        sp = jnp.zeros((C, C), jnp.float32)
        spt = jnp.zeros((C, C), jnp.float32)
        for _ in range(TK):
            m = jnp.max(w, axis=1, keepdims=True)
            am = jnp.min(jnp.where(w == m, cols, C), axis=1, keepdims=True)
            sel = cols == am
            sp = jnp.where(sel, m, sp)
            spt = jnp.where(am[:, 0][None, :] == rows, m[:, 0][None, :], spt)
            w = jnp.where(sel, NEGINF, w)
        r = jnp.maximum(sp, spt)
        out_ref[...] = jnp.where(diag, 0.0, r)


def _f32(shape):
    return jax.ShapeDtypeStruct(shape, jnp.float32)


def _full(shape):
    return pl.BlockSpec(shape, lambda i: tuple(0 for _ in shape))


def _blk(shape):
    return pl.BlockSpec(shape, lambda i: (i,) + tuple(0 for _ in shape[1:]))


def kernel(x, A_in, A_motif, coords, params):
    p = params
    f32 = jnp.float32
    tau = jnp.maximum(p['tau'], 0.1).astype(f32)
    mu_sp = jax.nn.softplus(p['mu']).astype(f32)
    wp = p['prune_w']
    wr = p['rewire_w']
    scal = jnp.stack([tau, wp[2 * H2], p['prune_b'], wr[2 * H2],
                      p['rewire_b'], mu_sp, 0.0, 0.0]).reshape(1, 8)
    wproj = jnp.stack([wp[:H2], wp[H2:2 * H2], wr[:H2], wr[H2:2 * H2]],
                      axis=0)

    vals8, idx8, hh, sd, sd_t, xw = pl.pallas_call(
        _topk_proj_body,
        grid=(NB,),
        in_specs=[_blk((BR, N)), _full((N, DIN)), _full((DIN, H2)),
                  _full((2, HID)), _full((2, HID)), _full((DIN, H2))],
        out_specs=(_blk((BR, TK)), _blk((BR, TK)), _full((N, H2)),
                   _full((N, 2)), _full((2, N)), _full((N, H2))),
        out_shape=(_f32((N, TK)), jax.ShapeDtypeStruct((N, TK), jnp.int32),
                   _f32((N, H2)), _f32((N, 2)), _f32((2, N)), _f32((N, H2))),
    )(A_motif, x, p['gat_W'], p['a_src'], p['a_dst'], p['gcn_W'])

    vals_t = vals8.T
    idx_t = idx8.T
    coords_t = coords.T

    gout, degm = pl.pallas_call(
        _gat_body,
        grid=(NB,),
        in_specs=[_blk((BR, N)), _full((N, H2)), _full((2, N)), _blk((BR, 2)),
                  _blk((BR, TK)), _blk((BR, TK)), _full((TK, N)),
                  _full((TK, N))],
        out_specs=(_blk((BR, H2)), _blk((BR, 1))),
        out_shape=(_f32((N, H2)), _f32((N, 1))),
    )(A_in, hh, sd_t, sd, vals8, idx8, vals_t, idx_t)

    gcn_pre = pl.pallas_call(
        _gcn_body,
        grid=(NB,),
        in_specs=[_blk((BR, TK)), _blk((BR, TK)), _full((TK, N)),
                  _full((TK, N)), _full((N, 1)), _blk((BR, 1)),
                  _full((N, H2))],
        out_specs=_blk((BR, H2)),
        out_shape=_f32((N, H2)),
    )(vals8, idx8, vals_t, idx_t, degm, degm, xw)

    h, vec4, hp = pl.pallas_call(
        _combine_body,
        out_shape=(_f32((N, H2)), _f32((N, 4)), _f32((N, C))),
    )(gout, gcn_pre, p['gat_b'].reshape(1, H2), p['gcn_b'].reshape(1, H2),
      p['bnA_g'].reshape(1, H2), p['bnA_b'].reshape(1, H2),
      p['bnM_g'].reshape(1, H2), p['bnM_b'].reshape(1, H2),
      scal, wproj, p['pool_W'])

    vec4_t = vec4.T

    thr = pl.pallas_call(
        _thr_body,
        grid=(NB,),
        in_specs=[_blk((BR, N)), _full((2, N)), _blk((BR, 2)),
                  _blk((BR, TK)), _blk((BR, TK)), _full((TK, N)),
                  _full((TK, N)), _full((4, N)), _blk((BR, 4)),
                  _full((1, 8))],
        out_specs=_blk((BR, 1)),
        out_shape=_f32((N, 1)),
    )(A_in, coords_t, coords, vals8, idx8, vals_t, idx_t, vec4_t, vec4, scal)

    thr_t = thr.T

    aref, degr = pl.pallas_call(
        _refine_body,
        grid=(NB,),
        in_specs=[_blk((BR, N)), _full((2, N)), _blk((BR, 2)),
                  _blk((BR, TK)), _blk((BR, TK)), _full((TK, N)),
                  _full((TK, N)), _full((4, N)), _blk((BR, 4)),
                  _full((1, 8)), _full((1, N)), _blk((BR, 1))],
        out_specs=(_blk((BR, N)), _blk((BR, 1))),
        out_shape=(_f32((N, N)), _f32((N, 1))),
    )(A_in, coords_t, coords, vals8, idx8, vals_t, idx_t, vec4_t, vec4, scal,
      thr_t, thr)

    s_mat, x_coarse = pl.pallas_call(
        _pool_body,
        grid=(NB,),
        in_specs=[_blk((BR, N)), _full((N, 1)), _blk((BR, 1)),
                  _full((N, C)), _blk((BR, C)), _blk((BR, H2)),
                  _full((1, C))],
        out_specs=(_blk((BR, C)), _full((C, H2))),
        out_shape=(_f32((N, C)), _f32((C, H2))),
    )(aref, degr, degr, hp, hp, h, p['pool_b'].reshape(1, C))

    a_coarse = pl.pallas_call(
        _coarse_body,
        grid=(NB,),
        in_specs=[_blk((BR, N)), _full((N, C)), _blk((BR, C))],
        out_specs=_full((C, C)),
        out_shape=_f32((C, C)),
        scratch_shapes=[pltpu.VMEM((C, C), jnp.float32)],
    )(aref, s_mat, s_mat)

    return x_coarse, a_coarse


# speedup vs baseline: 4.7492x; 1.0654x over previous
import jax
import jax.numpy as jnp
from jax.experimental import pallas as pl
from jax.experimental.pallas import tpu as pltpu

N = 2048
DIN = 128
HID = 64
H2 = 2 * HID
C = 64
TK = 8
BR = 512
NB = N // BR
NEGINF = float("-inf")


def _rows_cols(i0):
    rows = i0 + jax.lax.broadcasted_iota(jnp.int32, (BR, 1), 0)
    cols = jax.lax.broadcasted_iota(jnp.int32, (BR, N), 1)
    return rows, cols


def _mhat_tile(vals_b, idx_b, vals_t, idx_t, rows, cols):
    sp = jnp.zeros((BR, N), jnp.float32)
    spT = jnp.zeros((BR, N), jnp.float32)
    for k in range(TK):
        sp = sp + jnp.where(cols == idx_b[:, k][:, None],
                            vals_b[:, k][:, None], 0.0)
        spT = spT + jnp.where(idx_t[k:k + 1, :] == rows,
                              vals_t[k:k + 1, :], 0.0)
    m = jnp.maximum(sp, spT)
    return jnp.where(cols == rows, 0.0, m)


def _gate(x, tau):
    s = jax.nn.sigmoid(x / tau)
    return jnp.clip(s * 1.2 - 0.1, 0.0, 1.0)


def _rowmax_first_argmax(v, cols):
    m = jnp.max(v, axis=1, keepdims=True)
    am = jnp.min(jnp.where(v == m, cols, N), axis=1, keepdims=True)
    return m, am


def _topk_proj_body(am_ref, x_ref, gw_ref, asrc_ref, adst_ref, gcnw_ref,
                    vals_ref, idx_ref, hh_ref, sd_ref, sdt_ref, xw_ref):
    w = am_ref[...]
    cols = jax.lax.broadcasted_iota(jnp.int32, (BR, N), 1)
    vs, ins = [], []
    for _ in range(TK):
        m, am = _rowmax_first_argmax(w, cols)
        vs.append(m)
        ins.append(am)
        w = jnp.where(cols == am, NEGINF, w)
    vals_ref[...] = jnp.concatenate(vs, axis=1)
    idx_ref[...] = jnp.concatenate(ins, axis=1)

    @pl.when(pl.program_id(0) == 0)
    def _():
        xv = x_ref[...]
        hh = jnp.dot(xv, gw_ref[...], preferred_element_type=jnp.float32)
        hh_ref[...] = hh
        xw_ref[...] = jnp.dot(xv, gcnw_ref[...],
                              preferred_element_type=jnp.float32)
        dcols = []
        srows = []
        for hd in range(2):
            hhd = hh[:, hd * HID:(hd + 1) * HID]
            srows.append(jax.lax.dot_general(
                asrc_ref[hd:hd + 1, :], hhd, (((1,), (1,)), ((), ())),
                preferred_element_type=jnp.float32))
            dcols.append(jnp.dot(hhd, adst_ref[hd, :][:, None],
                                 preferred_element_type=jnp.float32))
        sd_ref[...] = jnp.concatenate(dcols, axis=1)
        sdt_ref[...] = jnp.concatenate(srows, axis=0)


def _gat_body(ain_ref, hh_ref, sdt_ref, sdb_ref, vb_ref, ib_ref, vt_ref,
              it_ref, gout_ref, degm_ref):
    rows, cols = _rows_cols(pl.program_id(0) * BR)
    m_hat = _mhat_tile(vb_ref[...], ib_ref[...], vt_ref[...], it_ref[...],
                       rows, cols)
    degm_ref[...] = jnp.sum(m_hat, axis=1, keepdims=True)
    a = ain_ref[...]
    adjb = (a > 0) | (cols == rows)
    hh = hh_ref[...]
    sdt = sdt_ref[...]
    sdb = sdb_ref[...]
    outs = []
    for hd in range(2):
        hhd = hh[:, hd * HID:(hd + 1) * HID]
        s = sdt[hd:hd + 1, :]
        d = sdb[:, hd][:, None]
        e = d + s
        e = jnp.where(e >= 0, e, 0.2 * e)
        e = jnp.where(adjb, e, -1e9)
        m = jnp.max(e, axis=1, keepdims=True)
        p = jnp.exp(e - m)
        alpha = p / jnp.sum(p, axis=1, keepdims=True)
        outs.append(jnp.dot(alpha, hhd, preferred_element_type=jnp.float32))
    gout_ref[...] = jnp.concatenate(outs, axis=1)


def _gcn_body(vb_ref, ib_ref, vf_ref, if_ref, degf_ref, degb_ref, xw_ref,
              out_ref):
    rows, cols = _rows_cols(pl.program_id(0) * BR)
    m = _mhat_tile(vb_ref[...], ib_ref[...], vf_ref[...], if_ref[...],
                   rows, cols)
    degf = degf_ref[...]
    dinv_f = jnp.where(degf > 0, jax.lax.rsqrt(jnp.where(degf > 0, degf, 1.0)),
                       0.0)
    degb = degb_ref[...]
    dinv_b = jnp.where(degb > 0, jax.lax.rsqrt(jnp.where(degb > 0, degb, 1.0)),
                       0.0)
    dxw = dinv_f * xw_ref[...]
    out_ref[...] = dinv_b * jnp.dot(m, dxw, preferred_element_type=jnp.float32)


def _bn_elu(v, g, b):
    mu = jnp.mean(v, axis=0, keepdims=True)
    var = jnp.mean((v - mu) * (v - mu), axis=0, keepdims=True)
    z = (v - mu) / jnp.sqrt(var + 1e-5) * g + b
    return jnp.where(z > 0, z, jnp.exp(z) - 1.0)


def _combine_body(gout_ref, gcn_ref, gatb_ref, gcnb_ref, bnag_ref, bnab_ref,
                  bnmg_ref, bnmb_ref, scal_ref, wproj_ref, poolw_ref,
                  h_ref, vec4_ref, hp_ref):
    h_a = _bn_elu(gout_ref[...] + gatb_ref[...], bnag_ref[...], bnab_ref[...])
    h_m = _bn_elu(gcn_ref[...] + gcnb_ref[...], bnmg_ref[...], bnmb_ref[...])
    h = h_a + scal_ref[0, 5] * h_m
    h_ref[...] = h
    vec4_ref[...] = jax.lax.dot_general(
        h, wproj_ref[...], (((1,), (1,)), ((), ())),
        preferred_element_type=jnp.float32)
    hp_ref[...] = jnp.dot(h, poolw_ref[...],
                          preferred_element_type=jnp.float32)


def _rewire_scores(a, coords_t, coords_b, m, vec4_t, vec4_b, rows, cols, scal):
    tau = scal[0, 0]
    w2r = scal[0, 3]
    rb = scal[0, 4]
    cxf = coords_t[0:1, :]
    cyf = coords_t[1:2, :]
    cxb = coords_b[:, 0][:, None]
    cyb = coords_b[:, 1][:, None]
    dist = jnp.abs(cxb - cxf) + jnp.abs(cyb - cyf)
    cand = (dist > 0) & (dist <= 2.0) & (a < 1e-6)
    arf = vec4_t[2:3, :]
    brf = vec4_t[3:4, :]
    arb = vec4_b[:, 2][:, None]
    brb = vec4_b[:, 3][:, None]
    base = m * w2r + rb
    l_ij = arb + brf + base
    l_ji = arf + brb + base
    return cand, jnp.where(rows < cols, l_ij, l_ji)


def _thr_body(ain_ref, coordsf_ref, coordsb_ref, vb_ref, ib_ref, vf_ref,
              if_ref, vec4f_ref, vec4b_ref, scal_ref, thr_ref):
    rows, cols = _rows_cols(pl.program_id(0) * BR)
    m = _mhat_tile(vb_ref[...], ib_ref[...], vf_ref[...], if_ref[...],
                   rows, cols)
    cand, lr = _rewire_scores(ain_ref[...], coordsf_ref[...], coordsb_ref[...],
                              m, vec4f_ref[...], vec4b_ref[...], rows, cols,
                              scal_ref[...])
    neg = jnp.where(cand, lr, NEGINF)
    m1, am = _rowmax_first_argmax(neg, cols)
    neg2 = jnp.where(cols == am, NEGINF, neg)
    l2 = jnp.max(neg2, axis=1, keepdims=True)
    thr_ref[...] = jnp.where(l2 == NEGINF, NEGINF, _gate(l2, scal_ref[0, 0]))


def _fused_tail_body(ain_ref, coordsf_ref, coordsb_ref, vb_ref, ib_ref,
                     vf_ref, if_ref, vec4f_ref, vec4b_ref, scal_ref,
                     thrf_ref, thrb_ref, hp_ref, h_ref, poolb_ref,
                     x_ref, out_ref, aref_s, deg_s, s_s, ac_s):
    ph = pl.program_id(0)
    j = pl.program_id(1)
    r0 = j * BR

    @pl.when(ph == 0)
    def _():
        rows, cols = _rows_cols(r0)
        a = ain_ref[...]
        scal = scal_ref[...]
        vec4_t = vec4f_ref[...]
        vec4_b = vec4b_ref[...]
        m = _mhat_tile(vb_ref[...], ib_ref[...], vf_ref[...], if_ref[...],
                       rows, cols)
        cand, lr = _rewire_scores(a, coordsf_ref[...], coordsb_ref[...], m,
                                  vec4_t, vec4_b, rows, cols, scal)
        zs = jnp.where(cand, _gate(lr, scal[0, 0]), 0.0)
        thr_b = thrb_ref[...]
        thr_f = thrf_ref[...]
        keep = cand & ((zs >= thr_b) | (zs >= thr_f))
        tau = scal[0, 0]
        w2p = scal[0, 1]
        pb = scal[0, 2]
        apf = vec4_t[0:1, :]
        bpf = vec4_t[1:2, :]
        apb = vec4_b[:, 0][:, None]
        bpb = vec4_b[:, 1][:, None]
        base = m * w2p + pb
        zp = _gate(jnp.where(rows < cols, apb + bpf + base,
                             apf + bpb + base), tau)
        aref = a * zp + 0.5 * jnp.where(keep, zs, 0.0)
        aref_s[pl.ds(r0, BR), :] = aref
        deg_s[pl.ds(r0, BR), :] = jnp.sum(aref, axis=1, keepdims=True) + 1.0

    @pl.when(ph == 1)
    def _():
        degf = deg_s[...]
        dinv_f = jnp.where(degf > 0,
                           jax.lax.rsqrt(jnp.where(degf > 0, degf, 1.0)), 0.0)
        degb = deg_s[pl.ds(r0, BR), :]
        dinv_b = jnp.where(degb > 0,
                           jax.lax.rsqrt(jnp.where(degb > 0, degb, 1.0)), 0.0)
        dhp = dinv_f * hp_ref[...]
        dhp_b = dinv_b * hp_ref[pl.ds(r0, BR), :]
        row = jnp.dot(aref_s[pl.ds(r0, BR), :], dhp,
                      preferred_element_type=jnp.float32)
        logits = dinv_b * (row + dhp_b) + poolb_ref[...]
        m = jnp.max(logits, axis=1, keepdims=True)
        p = jnp.exp(logits - m)
        s = p / jnp.sum(p, axis=1, keepdims=True)
        s_s[pl.ds(r0, BR), :] = s
        xc = jax.lax.dot_general(s, h_ref[pl.ds(r0, BR), :],
                                 (((0,), (0,)), ((), ())),
                                 preferred_element_type=jnp.float32)

        @pl.when(j == 0)
        def _():
            x_ref[...] = xc

        @pl.when(j != 0)
        def _():
            x_ref[...] += xc

    @pl.when(ph == 2)
    def _():
        t = jnp.dot(aref_s[pl.ds(r0, BR), :], s_s[...],
                    preferred_element_type=jnp.float32)
        acc = jax.lax.dot_general(s_s[pl.ds(r0, BR), :], t,
                                  (((0,), (0,)), ((), ())),
                                  preferred_element_type=jnp.float32)

        @pl.when(j == 0)
        def _():
            ac_s[...] = acc

        @pl.when(j != 0)
        def _():
            ac_s[...] += acc

    @pl.when((ph == 2) & (j == NB - 1))
    def _():
        rows = jax.lax.broadcasted_iota(jnp.int32, (C, 1), 0)
        cols = jax.lax.broadcasted_iota(jnp.int32, (C, C), 1)
        diag = cols == rows
        w = jnp.where(diag, 0.0, ac_s[...])
        sp = jnp.zeros((C, C), jnp.float32)
        spt = jnp.zeros((C, C), jnp.float32)
        for _ in range(TK):
            m = jnp.max(w, axis=1, keepdims=True)
            am = jnp.min(jnp.where(w == m, cols, C), axis=1, keepdims=True)
            sel = cols == am
            sp = jnp.where(sel, m, sp)
            spt = jnp.where(am[:, 0][None, :] == rows, m[:, 0][None, :], spt)
            w = jnp.where(sel, NEGINF, w)
        r = jnp.maximum(sp, spt)
        out_ref[...] = jnp.where(diag, 0.0, r)


def _f32(shape):
    return jax.ShapeDtypeStruct(shape, jnp.float32)


def _full(shape):
    return pl.BlockSpec(shape, lambda i: tuple(0 for _ in shape))


def _blk(shape):
    return pl.BlockSpec(shape, lambda i: (i,) + tuple(0 for _ in shape[1:]))


def kernel(x, A_in, A_motif, coords, params):
    p = params
    f32 = jnp.float32
    tau = jnp.maximum(p['tau'], 0.1).astype(f32)
    mu_sp = jax.nn.softplus(p['mu']).astype(f32)
    wp = p['prune_w']
    wr = p['rewire_w']
    scal = jnp.stack([tau, wp[2 * H2], p['prune_b'], wr[2 * H2],
                      p['rewire_b'], mu_sp, 0.0, 0.0]).reshape(1, 8)
    wproj = jnp.stack([wp[:H2], wp[H2:2 * H2], wr[:H2], wr[H2:2 * H2]],
                      axis=0)

    vals8, idx8, hh, sd, sd_t, xw = pl.pallas_call(
        _topk_proj_body,
        grid=(NB,),
        in_specs=[_blk((BR, N)), _full((N, DIN)), _full((DIN, H2)),
                  _full((2, HID)), _full((2, HID)), _full((DIN, H2))],
        out_specs=(_blk((BR, TK)), _blk((BR, TK)), _full((N, H2)),
                   _full((N, 2)), _full((2, N)), _full((N, H2))),
        out_shape=(_f32((N, TK)), jax.ShapeDtypeStruct((N, TK), jnp.int32),
                   _f32((N, H2)), _f32((N, 2)), _f32((2, N)), _f32((N, H2))),
    )(A_motif, x, p['gat_W'], p['a_src'], p['a_dst'], p['gcn_W'])

    vals_t = vals8.T
    idx_t = idx8.T
    coords_t = coords.T

    gout, degm = pl.pallas_call(
        _gat_body,
        grid=(NB,),
        in_specs=[_blk((BR, N)), _full((N, H2)), _full((2, N)), _blk((BR, 2)),
                  _blk((BR, TK)), _blk((BR, TK)), _full((TK, N)),
                  _full((TK, N))],
        out_specs=(_blk((BR, H2)), _blk((BR, 1))),
        out_shape=(_f32((N, H2)), _f32((N, 1))),
    )(A_in, hh, sd_t, sd, vals8, idx8, vals_t, idx_t)

    gcn_pre = pl.pallas_call(
        _gcn_body,
        grid=(NB,),
        in_specs=[_blk((BR, TK)), _blk((BR, TK)), _full((TK, N)),
                  _full((TK, N)), _full((N, 1)), _blk((BR, 1)),
                  _full((N, H2))],
        out_specs=_blk((BR, H2)),
        out_shape=_f32((N, H2)),
    )(vals8, idx8, vals_t, idx_t, degm, degm, xw)

    h, vec4, hp = pl.pallas_call(
        _combine_body,
        out_shape=(_f32((N, H2)), _f32((N, 4)), _f32((N, C))),
    )(gout, gcn_pre, p['gat_b'].reshape(1, H2), p['gcn_b'].reshape(1, H2),
      p['bnA_g'].reshape(1, H2), p['bnA_b'].reshape(1, H2),
      p['bnM_g'].reshape(1, H2), p['bnM_b'].reshape(1, H2),
      scal, wproj, p['pool_W'])

    vec4_t = vec4.T

    thr = pl.pallas_call(
        _thr_body,
        grid=(NB,),
        in_specs=[_blk((BR, N)), _full((2, N)), _blk((BR, 2)),
                  _blk((BR, TK)), _blk((BR, TK)), _full((TK, N)),
                  _full((TK, N)), _full((4, N)), _blk((BR, 4)),
                  _full((1, 8))],
        out_specs=_blk((BR, 1)),
        out_shape=_f32((N, 1)),
    )(A_in, coords_t, coords, vals8, idx8, vals_t, idx_t, vec4_t, vec4, scal)

    thr_t = thr.T

    def _full2(shape):
        return pl.BlockSpec(shape, lambda p, j: tuple(0 for _ in shape))

    def _blk2(shape):
        return pl.BlockSpec(
            shape, lambda p, j: (jnp.where(p == 0, j, 0),)
            + tuple(0 for _ in shape[1:]))

    x_coarse, a_coarse = pl.pallas_call(
        _fused_tail_body,
        grid=(3, NB),
        in_specs=[_blk2((BR, N)), _full2((2, N)), _blk2((BR, 2)),
                  _blk2((BR, TK)), _blk2((BR, TK)), _full2((TK, N)),
                  _full2((TK, N)), _full2((4, N)), _blk2((BR, 4)),
                  _full2((1, 8)), _full2((1, N)), _blk2((BR, 1)),
                  _full2((N, C)), _full2((N, H2)), _full2((1, C))],
        out_specs=(_full2((C, H2)), _full2((C, C))),
        out_shape=(_f32((C, H2)), _f32((C, C))),
        scratch_shapes=[pltpu.VMEM((N, N), jnp.float32),
                        pltpu.VMEM((N, 1), jnp.float32),
                        pltpu.VMEM((N, C), jnp.float32),
                        pltpu.VMEM((C, C), jnp.float32)],
    )(A_in, coords_t, coords, vals8, idx8, vals_t, idx_t, vec4_t, vec4, scal,
      thr_t, thr, hp, h, p['pool_b'].reshape(1, C))

    return x_coarse, a_coarse
